# baseline (speedup 1.0000x reference)
import jax
import jax.numpy as jnp
from jax import lax
from jax.experimental import pallas as pl
from jax.experimental.pallas import tpu as pltpu
from jax.experimental.pallas import tpu_sc as plsc

POOL_SIZE = 100000
POOL_DIM = 128
EPS = 1e-8

TC_ROWS = 74400
TC_BLOCK = 7440
TC_GRID = TC_ROWS // TC_BLOCK

_T_DIMS = (((1,), (1,)), ((), ()))
_N_DIMS = (((1,), (0,)), ((), ()))

L = 16
NK = POOL_DIM // L
NWORKERS = 32
SC_ROWS = POOL_SIZE - TC_ROWS
QUOTA = SC_ROWS // NWORKERS
CHUNK = 80

_MAGIC = 0x5F3759DF


def _tc_body(x_ref, mem_ref, out_ref, acc_ref):
    i = pl.program_id(0)
    x2 = x_ref[...]
    ones2 = jnp.ones((1, POOL_DIM), jnp.float32)
    xnsq = jnp.maximum(jnp.sum(x2 * x2), EPS * EPS)

    m = mem_ref[...]
    dots = jax.lax.dot_general(x2, m, _T_DIMS,
                               preferred_element_type=jnp.float32)
    nsq = jax.lax.dot_general(ones2, m * m, _T_DIMS,
                              preferred_element_type=jnp.float32)
    sims = dots * jax.lax.rsqrt(jnp.maximum(nsq, EPS * EPS) * xnsq)
    partial = jax.lax.dot_general(sims, m, _N_DIMS,
                                  preferred_element_type=jnp.float32)

    @pl.when(i == 0)
    def _():
        acc_ref[...] = jnp.zeros_like(acc_ref)

    acc_ref[...] += partial

    @pl.when(i == TC_GRID - 1)
    def _():
        out_ref[...] = acc_ref[...]


def _rsqrt16(a):
    i = plsc.bitcast(a, jnp.int32)
    i = jnp.int32(_MAGIC) - lax.shift_right_logical(i, 1)
    y = plsc.bitcast(i, jnp.float32)
    y = y * (1.5 - 0.5 * a * y * y)
    return y


def _row_update(buf, r, x_regs, xnsq_vec, acc):
    rv = [buf[r, pl.ds(k * L, L)] for k in range(NK)]
    dotv = rv[0] * x_regs[0]
    nv = rv[0] * rv[0]
    for k in range(1, NK):
        dotv = dotv + rv[k] * x_regs[k]
        nv = nv + rv[k] * rv[k]
    dotb = jnp.full((L,), jnp.sum(dotv), jnp.float32)
    nsqb = jnp.full((L,), jnp.sum(nv), jnp.float32)
    a = jnp.maximum(nsqb, EPS * EPS) * xnsq_vec
    sim = dotb * _rsqrt16(a)
    return [acc[k] + sim * rv[k] for k in range(NK)]


def _sc_body(x_hbm, mem_hbm, part_hbm, xv, buf0, buf1, accv, sem0, sem1, semx):
    wid = lax.axis_index("s") * 2 + lax.axis_index("c")
    base = TC_ROWS + wid * QUOTA
    nchunks = QUOTA // CHUNK

    pltpu.make_async_copy(x_hbm, xv, semx).start()
    pltpu.make_async_copy(x_hbm, xv, semx).wait()
    x_regs = [xv[pl.ds(k * L, L)] for k in range(NK)]
    xnv = x_regs[0] * x_regs[0]
    for k in range(1, NK):
        xnv = xnv + x_regs[k] * x_regs[k]
    xnsq_vec = jnp.maximum(jnp.full((L,), jnp.sum(xnv), jnp.float32),
                           EPS * EPS)

    def start_dma(c, buf, sem):
        pltpu.make_async_copy(mem_hbm.at[pl.ds(base + c * CHUNK, CHUNK), :],
                              buf, sem).start()

    def wait_dma(buf, sem):
        pltpu.make_async_copy(mem_hbm.at[pl.ds(0, CHUNK), :], buf, sem).wait()

    def compute_chunk(buf, acc):
        def row_block(rb, acc_t):
            acc_l = list(acc_t)
            rbase = rb * 8
            for j in range(8):
                acc_l = _row_update(buf, rbase + j, x_regs, xnsq_vec, acc_l)
            return tuple(acc_l)

        return list(lax.fori_loop(0, CHUNK // 8, row_block, tuple(acc)))

    start_dma(0, buf0, sem0)

    def pair_body(p, acc):
        c0 = p * 2
        start_dma(c0 + 1, buf1, sem1)
        wait_dma(buf0, sem0)
        acc = compute_chunk(buf0, list(acc))

        @pl.when(c0 + 2 < nchunks)
        def _():
            start_dma(c0 + 2, buf0, sem0)

        wait_dma(buf1, sem1)
        acc = compute_chunk(buf1, acc)
        return tuple(acc)

    acc0 = tuple(jnp.zeros((L,), jnp.float32) for _ in range(NK))
    acc = lax.fori_loop(0, nchunks // 2, pair_body, acc0)

    for k in range(NK):
        accv[pl.ds(k * L, L)] = acc[k]
    pltpu.make_async_copy(accv, part_hbm.at[wid], semx).start()
    pltpu.make_async_copy(accv, part_hbm.at[wid], semx).wait()


def _combine_body(sc_ref, tc_ref, out_ref):
    s = jnp.sum(sc_ref[...], axis=0) + tc_ref[0, :]
    out_ref[...] = (s / jnp.max(jnp.abs(s)))[None, :]


@jax.jit
def kernel(x, mem):
    x2 = x.reshape(1, POOL_DIM)
    mesh = plsc.VectorSubcoreMesh(core_axis_name="c", subcore_axis_name="s")
    sc_parts = pl.kernel(
        _sc_body,
        out_type=jax.ShapeDtypeStruct((NWORKERS, POOL_DIM), jnp.float32),
        mesh=mesh,
        compiler_params=pltpu.CompilerParams(needs_layout_passes=False, has_side_effects=False, skip_device_barrier=True),
        scratch_types=[
            pltpu.VMEM((POOL_DIM,), jnp.float32),
            pltpu.VMEM((CHUNK, POOL_DIM), jnp.float32),
            pltpu.VMEM((CHUNK, POOL_DIM), jnp.float32),
            pltpu.VMEM((POOL_DIM,), jnp.float32),
            pltpu.SemaphoreType.DMA,
            pltpu.SemaphoreType.DMA,
            pltpu.SemaphoreType.DMA,
        ],
    )(x, mem)
    tc_part = pl.pallas_call(
        _tc_body,
        grid=(TC_GRID,),
        in_specs=[
            pl.BlockSpec((1, POOL_DIM), lambda i: (0, 0)),
            pl.BlockSpec((TC_BLOCK, POOL_DIM), lambda i: (i, 0)),
        ],
        out_specs=pl.BlockSpec((1, POOL_DIM), lambda i: (0, 0)),
        out_shape=jax.ShapeDtypeStruct((1, POOL_DIM), jnp.float32),
        scratch_shapes=[pltpu.VMEM((1, POOL_DIM), jnp.float32)],
    )(x2, mem)
    out = pl.pallas_call(
        _combine_body,
        out_shape=jax.ShapeDtypeStruct((1, POOL_DIM), jnp.float32),
    )(sc_parts, tc_part)
    return out.reshape(POOL_DIM)


# speedup vs baseline: 1.7275x; 1.7275x over previous
import jax
import jax.numpy as jnp
from jax.experimental import pallas as pl
from jax.experimental.pallas import tpu as pltpu

POOL_SIZE = 100000
POOL_DIM = 128
EPS = 1e-8
BLOCK_ROWS = 20000
NUM_BLOCKS = POOL_SIZE // BLOCK_ROWS
_CHUNK_STARTS = tuple(range(0, BLOCK_ROWS - POOL_DIM, POOL_DIM)) + (BLOCK_ROWS - POOL_DIM,)
_OVERLAP = POOL_DIM - (BLOCK_ROWS - (BLOCK_ROWS // POOL_DIM) * POOL_DIM)

_T_DIMS = (((1,), (1,)), ((), ()))
_N_DIMS = (((1,), (0,)), ((), ()))


def _body(x_ref, mem_ref, out_ref, acc_ref):
    i = pl.program_id(0)
    x2 = x_ref[...]
    ones2 = jnp.ones((1, POOL_DIM), jnp.float32)
    xnsq = jnp.maximum(jnp.sum(x2 * x2), EPS * EPS)

    m = mem_ref[...]
    mb = m.astype(jnp.bfloat16)
    dots = jax.lax.dot_general(x2.astype(jnp.bfloat16), mb, _T_DIMS,
                               preferred_element_type=jnp.float32)
    nsq = jax.lax.dot_general(ones2.astype(jnp.bfloat16), mb * mb, _T_DIMS,
                              preferred_element_type=jnp.float32)
    sims = dots * jax.lax.rsqrt(jnp.maximum(nsq, EPS * EPS) * xnsq)
    partial = jax.lax.dot_general(sims.astype(jnp.bfloat16), mb, _N_DIMS,
                                  preferred_element_type=jnp.float32)

    @pl.when(i == 0)
    def _():
        acc_ref[...] = jnp.zeros_like(acc_ref)

    acc_ref[...] += partial

    @pl.when(i == NUM_BLOCKS - 1)
    def _():
        acc = acc_ref[...]
        out_ref[...] = acc / jnp.max(jnp.abs(acc))


def kernel(x, mem):
    out = pl.pallas_call(
        _body,
        grid=(NUM_BLOCKS,),
        in_specs=[
            pl.BlockSpec((1, POOL_DIM), lambda i: (0, 0)),
            pl.BlockSpec((BLOCK_ROWS, POOL_DIM), lambda i: (i, 0)),
        ],
        out_specs=pl.BlockSpec((1, POOL_DIM), lambda i: (0, 0)),
        out_shape=jax.ShapeDtypeStruct((1, POOL_DIM), jnp.float32),
        scratch_shapes=[pltpu.VMEM((1, POOL_DIM), jnp.float32)],
    )(x.reshape(1, POOL_DIM), mem)
    return out.reshape(POOL_DIM)
